# Initial kernel scaffold; baseline (speedup 1.0000x reference)
import jax
import jax.numpy as jnp
from jax.experimental import pallas as pl


def kernel(input_values, mask_time_indices, W, code_book):
    raise NotImplementedError("write your pallas kernel here")


# speedup vs baseline: 2.8472x; 2.8472x over previous
import jax
import jax.numpy as jnp
from jax.experimental import pallas as pl
from jax.experimental.pallas import tpu as pltpu

_B, _L, _D = 8, 2048, 320
_K, _NC = 16, 1024
_R = _B * _L
_BR = 512
_NB = _R // _BR


def _body(x_ref, mask_ref, wt_ref, ct_ref, cb_ref, out_ref, sval_ref, sint_ref):
    i = pl.program_id(0)

    x = x_ref[...]
    wt = wt_ref[...]
    ct = ct_ref[...]
    cb = cb_ref[...]

    t = jnp.dot(x, wt, preferred_element_type=jnp.float32)
    s = jnp.dot(t, ct, preferred_element_type=jnp.float32)
    cbsq = jnp.sum(ct * ct, axis=0, keepdims=True)
    adj = cbsq - 2.0 * s

    rowmin = jnp.min(adj, axis=1, keepdims=True)
    colio = jax.lax.broadcasted_iota(jnp.int32, (_BR, _NC), 1)
    rowarg = jnp.min(jnp.where(adj == rowmin, colio, _NC),
                     axis=1, keepdims=True)

    onehot = (colio == rowarg).astype(jnp.float32)
    cwin = jnp.dot(onehot, cb, preferred_element_type=jnp.float32)
    diff = t - cwin
    d2 = jnp.sum(diff * diff, axis=1, keepdims=True)

    maskv = mask_ref[...]
    vals = jnp.where(maskv == 1, d2, jnp.inf)

    rio = jax.lax.broadcasted_iota(jnp.int32, (_BR, 1), 0)
    bval = jnp.min(vals)
    bidx = jnp.min(jnp.where(vals == bval, rio, _R))
    bcol = jnp.sum(jnp.where(rio == bidx, rowarg, 0))
    rank_in = jnp.sum(jnp.where(rio <= bidx, maskv, 0))
    bcnt = jnp.sum(maskv)

    @pl.when(i == 0)
    def _init():
        sval_ref[0] = jnp.float32(jnp.inf)
        sint_ref[0] = jnp.sum(jnp.where(rio == 0, rowarg, 0))
        sint_ref[1] = -1
        sint_ref[2] = 0

    prev = sint_ref[2]

    @pl.when(bval < sval_ref[0])
    def _update():
        sval_ref[0] = bval
        sint_ref[0] = bcol
        sint_ref[1] = prev + rank_in - 1

    sint_ref[2] = prev + bcnt

    @pl.when(i == _NB - 1)
    def _finish():
        out_ref[0] = sint_ref[1] * _NC + sint_ref[0]


def kernel(input_values, mask_time_indices, W, code_book):
    x = input_values.reshape(_R, _D)
    mask2 = mask_time_indices.reshape(_R, 1)
    wt = W.T
    ct = code_book.T

    out = pl.pallas_call(
        _body,
        grid=(_NB,),
        in_specs=[
            pl.BlockSpec((_BR, _D), lambda i: (i, 0)),
            pl.BlockSpec((_BR, 1), lambda i: (i, 0)),
            pl.BlockSpec((_D, _K), lambda i: (0, 0)),
            pl.BlockSpec((_K, _NC), lambda i: (0, 0)),
            pl.BlockSpec((_NC, _K), lambda i: (0, 0)),
        ],
        out_specs=pl.BlockSpec(memory_space=pltpu.SMEM),
        out_shape=jax.ShapeDtypeStruct((1,), jnp.int32),
        scratch_shapes=[
            pltpu.SMEM((1,), jnp.float32),
            pltpu.SMEM((3,), jnp.int32),
        ],
    )(x, mask2, wt, ct, code_book)
    return out[0]


# speedup vs baseline: 6.5492x; 2.3002x over previous
import jax
import jax.numpy as jnp
from jax.experimental import pallas as pl
from jax.experimental.pallas import tpu as pltpu

_B, _L, _D = 8, 2048, 320
_K, _NC = 16, 1024


def _body(x_ref, mask_ref, w_ref, cb_ref, out_ref, sval_ref, sint_ref):
    i = pl.program_id(0)

    x = x_ref[0]
    w = w_ref[...]
    cb = cb_ref[...]

    t = jax.lax.dot_general(x, w, (((1,), (1,)), ((), ())),
                            preferred_element_type=jnp.float32)
    tt = t.T

    sqcb = cb * cb
    ones_k = jnp.ones((1, _K), dtype=jnp.float32)
    cbsq = jax.lax.dot_general(sqcb, ones_k, (((1,), (1,)), ((), ())),
                               preferred_element_type=jnp.float32)
    cb_aug = jnp.concatenate([cb * -2.0, cbsq], axis=1)
    t_aug = jnp.concatenate([tt, jnp.ones((1, _L), jnp.float32)], axis=0)
    adj = jnp.dot(cb_aug, t_aug, preferred_element_type=jnp.float32)

    rowmin = jnp.min(adj, axis=0, keepdims=True)
    codeio = jax.lax.broadcasted_iota(jnp.int32, (_NC, _L), 0)
    rowarg = jnp.min(jnp.where(adj == rowmin, codeio, _NC),
                     axis=0, keepdims=True)

    tsq = jnp.sum(tt * tt, axis=0, keepdims=True)
    d2 = tsq + rowmin

    mrow = mask_ref[pl.ds(i, 1), :]
    vals = jnp.where(mrow == 1, d2, jnp.inf)

    lio = jax.lax.broadcasted_iota(jnp.int32, (1, _L), 1)
    bval = jnp.min(vals)
    bidx = jnp.min(jnp.where(vals == bval, lio, _L))
    bcol = jnp.sum(jnp.where(lio == bidx, rowarg, 0))
    rank_in = jnp.sum(jnp.where(lio <= bidx, mrow, 0))
    bcnt = jnp.sum(mrow)

    @pl.when(i == 0)
    def _init():
        sval_ref[0] = jnp.float32(jnp.inf)
        sint_ref[0] = jnp.sum(jnp.where(lio == 0, rowarg, 0))
        sint_ref[1] = -1
        sint_ref[2] = 0

    prev = sint_ref[2]

    @pl.when(bval < sval_ref[0])
    def _update():
        sval_ref[0] = bval
        sint_ref[0] = bcol
        sint_ref[1] = prev + rank_in - 1

    sint_ref[2] = prev + bcnt

    @pl.when(i == _B - 1)
    def _finish():
        out_ref[0] = sint_ref[1] * _NC + sint_ref[0]


def kernel(input_values, mask_time_indices, W, code_book):
    out = pl.pallas_call(
        _body,
        grid=(_B,),
        in_specs=[
            pl.BlockSpec((1, _L, _D), lambda i: (i, 0, 0)),
            pl.BlockSpec((_B, _L), lambda i: (0, 0)),
            pl.BlockSpec((_K, _D), lambda i: (0, 0)),
            pl.BlockSpec((_NC, _K), lambda i: (0, 0)),
        ],
        out_specs=pl.BlockSpec(memory_space=pltpu.SMEM),
        out_shape=jax.ShapeDtypeStruct((1,), jnp.int32),
        scratch_shapes=[
            pltpu.SMEM((1,), jnp.float32),
            pltpu.SMEM((3,), jnp.int32),
        ],
    )(input_values, mask_time_indices, W, code_book)
    return out[0]


# speedup vs baseline: 13.5237x; 2.0650x over previous
import jax
import jax.numpy as jnp
from jax.experimental import pallas as pl
from jax.experimental.pallas import tpu as pltpu

_B, _L, _D = 8, 2048, 320
_K, _NC = 16, 1024


def _body(xt_ref, mask_ref, w_ref, cbt_ref, out_ref, sval_ref, sint_ref):
    i = pl.program_id(0)

    xt = xt_ref[0]
    w = w_ref[...]
    cbt = cbt_ref[...]

    tt = jnp.dot(w, xt, preferred_element_type=jnp.float32)

    cbsq = jnp.sum(cbt * cbt, axis=0, keepdims=True)
    cbt_aug = jnp.concatenate([cbt * -2.0, cbsq], axis=0)
    tt_aug = jnp.concatenate([tt, jnp.ones((1, _L), jnp.float32)], axis=0)
    adj = jax.lax.dot_general(cbt_aug, tt_aug, (((0,), (0,)), ((), ())),
                              preferred_element_type=jnp.float32)

    rowmin = jnp.min(adj, axis=0, keepdims=True)
    codeio = jax.lax.broadcasted_iota(jnp.int32, (_NC, _L), 0)
    rowarg = jnp.min(jnp.where(adj == rowmin, codeio, _NC),
                     axis=0, keepdims=True)

    tsq = jnp.sum(tt * tt, axis=0, keepdims=True)
    d2 = tsq + rowmin

    mrow = mask_ref[pl.ds(i, 1), :]
    vals = jnp.where(mrow == 1, d2, jnp.inf)

    lio = jax.lax.broadcasted_iota(jnp.int32, (1, _L), 1)
    bval = jnp.min(vals)
    bidx = jnp.min(jnp.where(vals == bval, lio, _L))
    bcol = jnp.sum(jnp.where(lio == bidx, rowarg, 0))
    rank_in = jnp.sum(jnp.where(lio <= bidx, mrow, 0))
    bcnt = jnp.sum(mrow)

    @pl.when(i == 0)
    def _init():
        sval_ref[0] = jnp.float32(jnp.inf)
        sint_ref[0] = jnp.sum(jnp.where(lio == 0, rowarg, 0))
        sint_ref[1] = -1
        sint_ref[2] = 0

    prev = sint_ref[2]

    @pl.when(bval < sval_ref[0])
    def _update():
        sval_ref[0] = bval
        sint_ref[0] = bcol
        sint_ref[1] = prev + rank_in - 1

    sint_ref[2] = prev + bcnt

    @pl.when(i == _B - 1)
    def _finish():
        out_ref[0] = sint_ref[1] * _NC + sint_ref[0]


def kernel(input_values, mask_time_indices, W, code_book):
    xt = input_values.transpose(0, 2, 1)
    cbt = code_book.T

    out = pl.pallas_call(
        _body,
        grid=(_B,),
        in_specs=[
            pl.BlockSpec((1, _D, _L), lambda i: (i, 0, 0)),
            pl.BlockSpec((_B, _L), lambda i: (0, 0)),
            pl.BlockSpec((_K, _D), lambda i: (0, 0)),
            pl.BlockSpec((_K, _NC), lambda i: (0, 0)),
        ],
        out_specs=pl.BlockSpec(memory_space=pltpu.SMEM),
        out_shape=jax.ShapeDtypeStruct((1,), jnp.int32),
        scratch_shapes=[
            pltpu.SMEM((1,), jnp.float32),
            pltpu.SMEM((3,), jnp.int32),
        ],
    )(xt, mask_time_indices, W, cbt)
    return out[0]


# speedup vs baseline: 16.1224x; 1.1922x over previous
import jax
import jax.numpy as jnp
from jax.experimental import pallas as pl
from jax.experimental.pallas import tpu as pltpu

_B, _L, _D = 8, 2048, 320
_K, _NC = 16, 1024
_CW = 128


def _phase1(xt_ref, mask_ref, w_ref, cbt_ref, out_ref, sval_ref, sint_ref):
    i = pl.program_id(0)

    xt = xt_ref[0]
    w = w_ref[...]
    cbt = cbt_ref[...]

    tt = jnp.dot(w, xt, preferred_element_type=jnp.float32)

    cbsq = jnp.sum(cbt * cbt, axis=0, keepdims=True)
    cbt_aug = jnp.concatenate([cbt * -2.0, cbsq], axis=0)
    tt_aug = jnp.concatenate([tt, jnp.ones((1, _L), jnp.float32)], axis=0)
    adj = jax.lax.dot_general(cbt_aug, tt_aug, (((0,), (0,)), ((), ())),
                              preferred_element_type=jnp.float32)

    rowmin = jnp.min(adj, axis=0, keepdims=True)
    tsq = jnp.sum(tt * tt, axis=0, keepdims=True)
    d2 = tsq + rowmin

    mrow = mask_ref[pl.ds(i, 1), :]
    vals = jnp.where(mrow == 1, d2, jnp.inf)

    lio = jax.lax.broadcasted_iota(jnp.int32, (1, _L), 1)
    bval = jnp.min(vals)
    bidx = jnp.min(jnp.where(vals == bval, lio, _L))
    rank_in = jnp.sum(jnp.where(lio <= bidx, mrow, 0))
    bcnt = jnp.sum(mrow)

    @pl.when(i == 0)
    def _init():
        sval_ref[0] = jnp.float32(jnp.inf)
        sint_ref[0] = 0
        sint_ref[1] = -1
        sint_ref[2] = 0

    prev = sint_ref[2]

    @pl.when(bval < sval_ref[0])
    def _update():
        sval_ref[0] = bval
        sint_ref[0] = i * _L + bidx
        sint_ref[1] = prev + rank_in - 1

    sint_ref[2] = prev + bcnt

    @pl.when(i == _B - 1)
    def _finish():
        out_ref[0] = sint_ref[0]
        out_ref[1] = sint_ref[1]


def _phase2(s_ref, xw_ref, w_ref, cbt_ref, out_ref):
    xw = xw_ref[0]
    w = w_ref[...]
    cbt = cbt_ref[...]

    tt = jnp.dot(w, xw, preferred_element_type=jnp.float32)

    cbsq = jnp.sum(cbt * cbt, axis=0, keepdims=True)
    cbt_aug = jnp.concatenate([cbt * -2.0, cbsq], axis=0)
    tt_aug = jnp.concatenate([tt, jnp.ones((1, _CW), jnp.float32)], axis=0)
    adj = jax.lax.dot_general(cbt_aug, tt_aug, (((0,), (0,)), ((), ())),
                              preferred_element_type=jnp.float32)

    rowmin = jnp.min(adj, axis=0, keepdims=True)
    codeio = jax.lax.broadcasted_iota(jnp.int32, (_NC, _CW), 0)
    rowarg = jnp.min(jnp.where(adj == rowmin, codeio, _NC),
                     axis=0, keepdims=True)

    lane = s_ref[0] % _CW
    lio = jax.lax.broadcasted_iota(jnp.int32, (1, _CW), 1)
    bcol = jnp.sum(jnp.where(lio == lane, rowarg, 0))
    out_ref[0] = s_ref[1] * _NC + bcol


def kernel(input_values, mask_time_indices, W, code_book):
    xt = input_values.transpose(0, 2, 1)
    cbt = code_book.T

    winner = pl.pallas_call(
        _phase1,
        grid=(_B,),
        in_specs=[
            pl.BlockSpec((1, _D, _L), lambda i: (i, 0, 0)),
            pl.BlockSpec((_B, _L), lambda i: (0, 0)),
            pl.BlockSpec((_K, _D), lambda i: (0, 0)),
            pl.BlockSpec((_K, _NC), lambda i: (0, 0)),
        ],
        out_specs=pl.BlockSpec(memory_space=pltpu.SMEM),
        out_shape=jax.ShapeDtypeStruct((2,), jnp.int32),
        scratch_shapes=[
            pltpu.SMEM((1,), jnp.float32),
            pltpu.SMEM((3,), jnp.int32),
        ],
    )(xt, mask_time_indices, W, cbt)

    out = pl.pallas_call(
        _phase2,
        grid_spec=pltpu.PrefetchScalarGridSpec(
            num_scalar_prefetch=1,
            grid=(1,),
            in_specs=[
                pl.BlockSpec((1, _D, _CW),
                             lambda i, s: (s[0] // _L, 0, (s[0] % _L) // _CW)),
                pl.BlockSpec((_K, _D), lambda i, s: (0, 0)),
                pl.BlockSpec((_K, _NC), lambda i, s: (0, 0)),
            ],
            out_specs=pl.BlockSpec(memory_space=pltpu.SMEM),
        ),
        out_shape=jax.ShapeDtypeStruct((1,), jnp.int32),
    )(winner, xt, W, cbt)
    return out[0]


# speedup vs baseline: 16.1834x; 1.0038x over previous
import jax
import jax.numpy as jnp
from jax.experimental import pallas as pl
from jax.experimental.pallas import tpu as pltpu

_B, _L, _D = 8, 2048, 320
_K, _NC = 16, 1024
_CW = 128


def _phase1(xa_ref, xb_ref, mask_ref, w_ref, cbt_ref, out_ref, sval_ref, sint_ref):
    i = pl.program_id(0)

    w = w_ref[...]
    cbt = cbt_ref[...]

    tta = jnp.dot(w, xa_ref[0], preferred_element_type=jnp.float32)
    ttb = jnp.dot(w, xb_ref[0], preferred_element_type=jnp.float32)
    tt = jnp.concatenate([tta, ttb], axis=1)

    cbsq = jnp.sum(cbt * cbt, axis=0, keepdims=True)
    cbt_aug = jnp.concatenate([cbt * -2.0, cbsq], axis=0)
    tt_aug = jnp.concatenate([tt, jnp.ones((1, _L), jnp.float32)], axis=0)
    adj = jax.lax.dot_general(cbt_aug, tt_aug, (((0,), (0,)), ((), ())),
                              preferred_element_type=jnp.float32)

    rowmin = jnp.min(adj, axis=0, keepdims=True)
    tsq = jnp.sum(tt * tt, axis=0, keepdims=True)
    d2 = tsq + rowmin

    mrow = mask_ref[pl.ds(i, 1), :]
    vals = jnp.where(mrow == 1, d2, jnp.inf)

    lio = jax.lax.broadcasted_iota(jnp.int32, (1, _L), 1)
    bval = jnp.min(vals)
    bidx = jnp.min(jnp.where(vals == bval, lio, _L))
    rank_in = jnp.sum(jnp.where(lio <= bidx, mrow, 0))
    bcnt = jnp.sum(mrow)

    @pl.when(i == 0)
    def _init():
        sval_ref[0] = jnp.float32(jnp.inf)
        sint_ref[0] = 0
        sint_ref[1] = -1
        sint_ref[2] = 0

    prev = sint_ref[2]

    @pl.when(bval < sval_ref[0])
    def _update():
        sval_ref[0] = bval
        sint_ref[0] = i * _L + bidx
        sint_ref[1] = prev + rank_in - 1

    sint_ref[2] = prev + bcnt

    @pl.when(i == _B - 1)
    def _finish():
        out_ref[0] = sint_ref[0]
        out_ref[1] = sint_ref[1]


def _phase2(s_ref, xw_ref, w_ref, cbt_ref, out_ref):
    xw = xw_ref[0]
    w = w_ref[...]
    cbt = cbt_ref[...]

    tt = jnp.dot(w, xw, preferred_element_type=jnp.float32)

    cbsq = jnp.sum(cbt * cbt, axis=0, keepdims=True)
    cbt_aug = jnp.concatenate([cbt * -2.0, cbsq], axis=0)
    tt_aug = jnp.concatenate([tt, jnp.ones((1, _CW), jnp.float32)], axis=0)
    adj = jax.lax.dot_general(cbt_aug, tt_aug, (((0,), (0,)), ((), ())),
                              preferred_element_type=jnp.float32)

    rowmin = jnp.min(adj, axis=0, keepdims=True)
    codeio = jax.lax.broadcasted_iota(jnp.int32, (_NC, _CW), 0)
    rowarg = jnp.min(jnp.where(adj == rowmin, codeio, _NC),
                     axis=0, keepdims=True)

    lane = s_ref[0] % _CW
    lio = jax.lax.broadcasted_iota(jnp.int32, (1, _CW), 1)
    bcol = jnp.sum(jnp.where(lio == lane, rowarg, 0))
    out_ref[0] = s_ref[1] * _NC + bcol


def kernel(input_values, mask_time_indices, W, code_book):
    xt = input_values.transpose(0, 2, 1)
    cbt = code_book.T

    winner = pl.pallas_call(
        _phase1,
        grid=(_B,),
        in_specs=[
            pl.BlockSpec((1, _D, _L // 2), lambda i: (i, 0, 0)),
            pl.BlockSpec((1, _D, _L // 2), lambda i: (i, 0, 1)),
            pl.BlockSpec((_B, _L), lambda i: (0, 0)),
            pl.BlockSpec((_K, _D), lambda i: (0, 0)),
            pl.BlockSpec((_K, _NC), lambda i: (0, 0)),
        ],
        out_specs=pl.BlockSpec(memory_space=pltpu.SMEM),
        out_shape=jax.ShapeDtypeStruct((2,), jnp.int32),
        scratch_shapes=[
            pltpu.SMEM((1,), jnp.float32),
            pltpu.SMEM((3,), jnp.int32),
        ],
    )(xt, xt, mask_time_indices, W, cbt)

    out = pl.pallas_call(
        _phase2,
        grid_spec=pltpu.PrefetchScalarGridSpec(
            num_scalar_prefetch=1,
            grid=(1,),
            in_specs=[
                pl.BlockSpec((1, _D, _CW),
                             lambda i, s: (s[0] // _L, 0, (s[0] % _L) // _CW)),
                pl.BlockSpec((_K, _D), lambda i, s: (0, 0)),
                pl.BlockSpec((_K, _NC), lambda i, s: (0, 0)),
            ],
            out_specs=pl.BlockSpec(memory_space=pltpu.SMEM),
        ),
        out_shape=jax.ShapeDtypeStruct((1,), jnp.int32),
    )(winner, xt, W, cbt)
    return out[0]


# speedup vs baseline: 17.6243x; 1.0890x over previous
import jax
import jax.numpy as jnp
from jax.experimental import pallas as pl
from jax.experimental.pallas import tpu as pltpu

_B, _L, _D = 8, 2048, 320
_K, _NC = 16, 1024


def _body(xt_ref, mask_ref, w_ref, cbt_ref, out_ref, sval_ref, sint_ref,
          adj_ref):
    i = pl.program_id(0)

    w = w_ref[...]
    cbt = cbt_ref[...]

    tt = jnp.dot(w, xt_ref[0], preferred_element_type=jnp.float32)

    cbsq = jnp.sum(cbt * cbt, axis=0, keepdims=True)
    cbt_aug = jnp.concatenate([cbt * -2.0, cbsq], axis=0)
    tt_aug = jnp.concatenate([tt, jnp.ones((1, _L), jnp.float32)], axis=0)
    adj_ref[...] = jax.lax.dot_general(
        cbt_aug, tt_aug, (((0,), (0,)), ((), ())),
        preferred_element_type=jnp.float32)

    rowmin = jnp.min(adj_ref[...], axis=0, keepdims=True)
    tsq = jnp.sum(tt * tt, axis=0, keepdims=True)
    d2 = tsq + rowmin

    mrow = mask_ref[pl.ds(i, 1), :]
    vals = jnp.where(mrow == 1, d2, jnp.inf)

    lio = jax.lax.broadcasted_iota(jnp.int32, (1, _L), 1)
    bval = jnp.min(vals)
    bidx = jnp.min(jnp.where(vals == bval, lio, _L))
    rank_in = jnp.sum(jnp.where(lio <= bidx, mrow, 0))
    bcnt = jnp.sum(mrow)

    cio = jax.lax.broadcasted_iota(jnp.int32, (_NC, 128), 0)
    wio = jax.lax.broadcasted_iota(jnp.int32, (_NC, 128), 1)

    def _col_argmin(tok):
        base = pl.multiple_of((tok // 128) * 128, 128)
        win = adj_ref[:, pl.ds(base, 128)]
        col = jnp.where(wio == tok % 128, win, jnp.inf)
        cmin = jnp.min(col)
        return jnp.min(jnp.where(col == cmin, cio, _NC))

    @pl.when(i == 0)
    def _init():
        sval_ref[0] = jnp.float32(jnp.inf)
        sint_ref[0] = _col_argmin(0)
        sint_ref[1] = -1
        sint_ref[2] = 0

    prev = sint_ref[2]

    @pl.when(bval < sval_ref[0])
    def _update():
        sval_ref[0] = bval
        sint_ref[0] = _col_argmin(bidx)
        sint_ref[1] = prev + rank_in - 1

    sint_ref[2] = prev + bcnt

    @pl.when(i == _B - 1)
    def _finish():
        out_ref[0] = sint_ref[1] * _NC + sint_ref[0]


def kernel(input_values, mask_time_indices, W, code_book):
    xt = input_values.transpose(0, 2, 1)
    cbt = code_book.T

    out = pl.pallas_call(
        _body,
        grid=(_B,),
        in_specs=[
            pl.BlockSpec((1, _D, _L), lambda i: (i, 0, 0)),
            pl.BlockSpec((_B, _L), lambda i: (0, 0)),
            pl.BlockSpec((_K, _D), lambda i: (0, 0)),
            pl.BlockSpec((_K, _NC), lambda i: (0, 0)),
        ],
        out_specs=pl.BlockSpec(memory_space=pltpu.SMEM),
        out_shape=jax.ShapeDtypeStruct((1,), jnp.int32),
        scratch_shapes=[
            pltpu.SMEM((1,), jnp.float32),
            pltpu.SMEM((3,), jnp.int32),
            pltpu.VMEM((_NC, _L), jnp.float32),
        ],
    )(xt, mask_time_indices, W, cbt)
    return out[0]
